# baseline (speedup 1.0000x reference)
import jax
import jax.numpy as jnp
from jax import lax
from jax.experimental import pallas as pl
from jax.experimental.pallas import tpu as pltpu
from jax.experimental.pallas import tpu_sc as plsc

_NUM_CORES = 2
_NUM_SUBCORES = 16
_NUM_WORKERS = _NUM_CORES * _NUM_SUBCORES
_CHUNK = 128
_WIDE = 128
_TBLOCK = 512


def _repack_block(t_ref, o_ref):
    o_ref[:, 0 : t_ref.shape[0]] = t_ref[...].T


def _repack_table(table_t, padded_rows):
    embed_dim = table_t.shape[0]
    grid = padded_rows // _TBLOCK
    return pl.pallas_call(
        _repack_block,
        grid=(grid,),
        in_specs=[
            pl.BlockSpec((embed_dim, _TBLOCK), lambda i: (0, i)),
        ],
        out_specs=pl.BlockSpec((_TBLOCK, _WIDE), lambda i: (i, 0)),
        out_shape=jax.ShapeDtypeStruct((padded_rows, _WIDE), table_t.dtype),
        compiler_params=pltpu.CompilerParams(
            dimension_semantics=("parallel",),
        ),
    )(table_t)


def kernel(scentences, table):
    batch, seq = scentences.shape
    num_indices = batch * seq
    embed_dim = table.shape[1]
    per_worker = num_indices // _NUM_WORKERS
    nchunks = per_worker // _CHUNK

    indices = scentences.reshape(num_indices).astype(jnp.int32)
    padded_rows = -(-table.shape[0] // _TBLOCK) * _TBLOCK
    table_pad = _repack_table(table.T, padded_rows)

    mesh = plsc.VectorSubcoreMesh(
        core_axis_name="core", subcore_axis_name="subcore"
    )

    @pl.kernel(
        out_type=jax.ShapeDtypeStruct((num_indices, _WIDE), table.dtype),
        mesh=mesh,
        compiler_params=pltpu.CompilerParams(use_tc_tiling_on_sc=False),
        scratch_types=[
            pltpu.VMEM((per_worker,), jnp.int32),
            pltpu.VMEM((2, _CHUNK, _WIDE), jnp.float32),
            pltpu.SemaphoreType.DMA,
            pltpu.SemaphoreType.DMA,
            pltpu.SemaphoreType.DMA,
            pltpu.SemaphoreType.DMA,
        ],
    )
    def gather_kernel(
        table_hbm, idx_hbm, out_hbm,
        idx_v, buf_v, gsem0, gsem1, osem0, osem1,
    ):
        wid = lax.axis_index("subcore") * _NUM_CORES + lax.axis_index("core")
        base = wid * per_worker
        gsems = (gsem0, gsem1)
        osems = (osem0, osem1)
        pltpu.sync_copy(idx_hbm.at[pl.ds(base, per_worker)], idx_v)

        def fire_gather(c, b):
            return pltpu.async_copy(
                table_hbm.at[idx_v.at[pl.ds(c * _CHUNK, _CHUNK)]],
                buf_v.at[b],
                gsems[b],
            )

        def fire_out(c, b):
            return pltpu.async_copy(
                buf_v.at[b],
                out_hbm.at[pl.ds(base + c * _CHUNK, _CHUNK)],
                osems[b],
            )

        def wait_gather(b):
            pltpu.make_async_copy(
                table_hbm.at[idx_v.at[pl.ds(0, _CHUNK)]],
                buf_v.at[b],
                gsems[b],
            ).wait()

        def wait_out(b):
            pltpu.make_async_copy(
                buf_v.at[b],
                out_hbm.at[pl.ds(base, _CHUNK)],
                osems[b],
            ).wait()

        fire_gather(0, 0)

        @pl.loop(0, nchunks, step=2)
        def _(c0):
            for b in range(2):
                c = c0 + b
                nb = 1 - b
                @pl.when(c + 1 < nchunks)
                def _():
                    @pl.when(c >= 1)
                    def _():
                        wait_out(nb)
                    fire_gather(c + 1, nb)

                wait_gather(b)
                fire_out(c, b)

        for b in range(2):
            wait_out(b)

    out = gather_kernel(table_pad, indices)
    return out[:, :embed_dim].reshape(batch, seq, embed_dim)


# speedup vs baseline: 2.0730x; 2.0730x over previous
import jax
import jax.numpy as jnp
from jax import lax
from jax.experimental import pallas as pl
from jax.experimental.pallas import tpu as pltpu
from jax.experimental.pallas import tpu_sc as plsc

_NUM_CORES = 2
_NUM_SUBCORES = 16
_NUM_WORKERS = _NUM_CORES * _NUM_SUBCORES
_CHUNK = 128
_WIDE = 128
_TBLOCK = 512


def kernel(scentences, table):
    batch, seq = scentences.shape
    num_indices = batch * seq
    embed_dim = table.shape[1]
    per_worker = num_indices // _NUM_WORKERS
    nchunks = per_worker // _CHUNK

    indices = scentences.reshape(num_indices).astype(jnp.int32)
    selector = jnp.eye(embed_dim, _WIDE, dtype=table.dtype)
    table_pad = jax.lax.dot_general(
        table,
        selector,
        (((1,), (0,)), ((), ())),
        precision=jax.lax.Precision.HIGHEST,
    )

    mesh = plsc.VectorSubcoreMesh(
        core_axis_name="core", subcore_axis_name="subcore"
    )

    @pl.kernel(
        out_type=jax.ShapeDtypeStruct((num_indices, _WIDE), table.dtype),
        mesh=mesh,
        compiler_params=pltpu.CompilerParams(use_tc_tiling_on_sc=False),
        scratch_types=[
            pltpu.VMEM((per_worker,), jnp.int32),
            pltpu.VMEM((2, _CHUNK, _WIDE), jnp.float32),
            pltpu.SemaphoreType.DMA,
            pltpu.SemaphoreType.DMA,
            pltpu.SemaphoreType.DMA,
            pltpu.SemaphoreType.DMA,
        ],
    )
    def gather_kernel(
        table_hbm, idx_hbm, out_hbm,
        idx_v, buf_v, gsem0, gsem1, osem0, osem1,
    ):
        wid = lax.axis_index("subcore") * _NUM_CORES + lax.axis_index("core")
        base = wid * per_worker
        gsems = (gsem0, gsem1)
        osems = (osem0, osem1)
        pltpu.sync_copy(idx_hbm.at[pl.ds(base, per_worker)], idx_v)

        def fire_gather(c, b):
            return pltpu.async_copy(
                table_hbm.at[idx_v.at[pl.ds(c * _CHUNK, _CHUNK)]],
                buf_v.at[b],
                gsems[b],
            )

        def fire_out(c, b):
            return pltpu.async_copy(
                buf_v.at[b],
                out_hbm.at[pl.ds(base + c * _CHUNK, _CHUNK)],
                osems[b],
            )

        def wait_gather(b):
            pltpu.make_async_copy(
                table_hbm.at[idx_v.at[pl.ds(0, _CHUNK)]],
                buf_v.at[b],
                gsems[b],
            ).wait()

        def wait_out(b):
            pltpu.make_async_copy(
                buf_v.at[b],
                out_hbm.at[pl.ds(base, _CHUNK)],
                osems[b],
            ).wait()

        fire_gather(0, 0)

        @pl.loop(0, nchunks, step=2)
        def _(c0):
            for b in range(2):
                c = c0 + b
                nb = 1 - b
                @pl.when(c + 1 < nchunks)
                def _():
                    @pl.when(c >= 1)
                    def _():
                        wait_out(nb)
                    fire_gather(c + 1, nb)

                wait_gather(b)
                fire_out(c, b)

        for b in range(2):
            wait_out(b)

    out = gather_kernel(table_pad, indices)
    return out[:, :embed_dim].reshape(batch, seq, embed_dim)
